# baseline (speedup 1.0000x reference)
import functools

import jax
import jax.numpy as jnp
from jax import lax
from jax.experimental import pallas as pl
from jax.experimental.pallas import tpu as pltpu
from jax.experimental.pallas import tpu_sc as plsc

N = 10000
E = 320000
G = 64

NC = 2
NS = 16
NW = NC * NS
EPW = E // NW
CHUNK = 125
NCHUNK = EPW // CHUNK
NBUF = 8
LEAD = 4
NGRP = NCHUNK // NBUF
RZ = 624
RTAIL = N - NS * RZ

ROWS = 2000
GRID = N // ROWS


def _edge_agg(table, src, dst, zeros, W):
    mesh = plsc.VectorSubcoreMesh(core_axis_name="c", subcore_axis_name="s")

    @functools.partial(
        pl.kernel,
        out_type=jax.ShapeDtypeStruct((NC, N, W), jnp.float32),
        mesh=mesh,
        scratch_types=[
            pltpu.VMEM((NCHUNK, CHUNK), jnp.int32),
            pltpu.VMEM((NCHUNK, CHUNK), jnp.int32),
            [pltpu.VMEM((CHUNK, W), jnp.float32) for _ in range(NBUF)],
            pltpu.VMEM_SHARED((N, W), jnp.float32),
            [pltpu.SemaphoreType.DMA for _ in range(NBUF)],
            [pltpu.SemaphoreType.DMA for _ in range(NBUF)],
        ],
        compiler_params=pltpu.CompilerParams(use_tc_tiling_on_sc=False),
    )
    def k(tab_hbm, src_hbm, dst_hbm, zero_hbm, out_hbm, sidx, didx, rows,
          acc, gsem, ssem):
        c = lax.axis_index("c")
        s = lax.axis_index("s")
        wid = c * NS + s

        pltpu.sync_copy(src_hbm.at[wid], sidx)
        pltpu.sync_copy(dst_hbm.at[wid], didx)
        for b in range(LEAD):
            pltpu.async_copy(tab_hbm.at[sidx.at[b]], rows[b], gsem[b])

        pltpu.sync_copy(zero_hbm.at[pl.ds(s * RZ, RZ)], acc.at[pl.ds(s * RZ, RZ)])

        @pl.when(s == NS - 1)
        def _():
            pltpu.sync_copy(zero_hbm.at[pl.ds(NS * RZ, RTAIL)],
                            acc.at[pl.ds(NS * RZ, RTAIL)])

        plsc.subcore_barrier()

        def body(g, carry):
            for b in range(NBUF):
                kk = g * NBUF + b
                pltpu.make_async_copy(tab_hbm.at[sidx.at[kk]], rows[b],
                                      gsem[b]).wait()
                pltpu.async_copy(rows[b], acc.at[didx.at[kk]], ssem[b],
                                 add=True)
                bl = (b + LEAD) % NBUF

                @pl.when(kk + LEAD < NCHUNK)
                def _(kk=kk, bl=bl):
                    @pl.when(kk - (NBUF - LEAD) >= 0)
                    def _():
                        kd = kk - (NBUF - LEAD)
                        pltpu.make_async_copy(rows[bl], acc.at[didx.at[kd]],
                                              ssem[bl]).wait()
                    pltpu.async_copy(tab_hbm.at[sidx.at[kk + LEAD]], rows[bl],
                                     gsem[bl])

            return carry

        lax.fori_loop(0, NGRP, body, 0)
        for b in range(NBUF):
            kd = NCHUNK - NBUF + b
            pltpu.make_async_copy(rows[b], acc.at[didx.at[kd]],
                                  ssem[b]).wait()
        plsc.subcore_barrier()
        pltpu.sync_copy(acc.at[pl.ds(s * RZ, RZ)], out_hbm.at[c, pl.ds(s * RZ, RZ)])

        @pl.when(s == NS - 1)
        def _():
            pltpu.sync_copy(acc.at[pl.ds(NS * RZ, RTAIL)],
                            out_hbm.at[c, pl.ds(NS * RZ, RTAIL)])

    return k(table, src, dst, zeros)


def _proj2(x, Wa, Wb):
    D = x.shape[1]
    O = Wa.shape[1]

    def body(x_ref, wa_ref, wb_ref, p_ref, r_ref):
        xb = x_ref[...]
        p_ref[...] = jnp.dot(xb, wa_ref[...], preferred_element_type=jnp.float32)
        r_ref[...] = jnp.dot(xb, wb_ref[...], preferred_element_type=jnp.float32)

    return pl.pallas_call(
        body,
        grid=(GRID,),
        in_specs=[
            pl.BlockSpec((ROWS, D), lambda i: (i, 0)),
            pl.BlockSpec((D, O), lambda i: (0, 0)),
            pl.BlockSpec((D, O), lambda i: (0, 0)),
        ],
        out_specs=[
            pl.BlockSpec((ROWS, O), lambda i: (i, 0)),
            pl.BlockSpec((ROWS, O), lambda i: (i, 0)),
        ],
        out_shape=[
            jax.ShapeDtypeStruct((N, O), jnp.float32),
            jax.ShapeDtypeStruct((N, O), jnp.float32),
        ],
    )(x, Wa, Wb)


def _elu(v):
    return jnp.where(v > 0, v, jnp.exp(jnp.minimum(v, 0.0)) - 1.0)


def _combine1(agg, r1, b1):
    W = r1.shape[1]

    def body(a_ref, r_ref, b_ref, o_ref):
        v = a_ref[0] + a_ref[1] + r_ref[...] + b_ref[...]
        o_ref[...] = _elu(v)

    return pl.pallas_call(
        body,
        grid=(GRID,),
        in_specs=[
            pl.BlockSpec((2, ROWS, W), lambda i: (0, i, 0)),
            pl.BlockSpec((ROWS, W), lambda i: (i, 0)),
            pl.BlockSpec((1, W), lambda i: (0, 0)),
        ],
        out_specs=pl.BlockSpec((ROWS, W), lambda i: (i, 0)),
        out_shape=jax.ShapeDtypeStruct((N, W), jnp.float32),
    )(agg, r1, b1)


def _combine_mm(agg, h, W_rel, b_rel, W_root):
    Wi = h.shape[1]
    Wo = W_rel.shape[1]

    def body(a_ref, h_ref, wr_ref, b_ref, wt_ref, o_ref):
        a = a_ref[0] + a_ref[1]
        v = (jnp.dot(a, wr_ref[...], preferred_element_type=jnp.float32)
             + jnp.dot(h_ref[...], wt_ref[...], preferred_element_type=jnp.float32)
             + b_ref[...])
        o_ref[...] = _elu(v)

    return pl.pallas_call(
        body,
        grid=(GRID,),
        in_specs=[
            pl.BlockSpec((2, ROWS, Wi), lambda i: (0, i, 0)),
            pl.BlockSpec((ROWS, Wi), lambda i: (i, 0)),
            pl.BlockSpec((Wi, Wo), lambda i: (0, 0)),
            pl.BlockSpec((1, Wo), lambda i: (0, 0)),
            pl.BlockSpec((Wi, Wo), lambda i: (0, 0)),
        ],
        out_specs=pl.BlockSpec((ROWS, Wo), lambda i: (i, 0)),
        out_shape=jax.ShapeDtypeStruct((N, Wo), jnp.float32),
    )(agg, h, W_rel, b_rel, W_root)


def _final_pool_head(agg, h2, W_rel, b_rel, W_root, batch3d,
                     fc1_W, fc1_b, fc2_W, fc2_b, fc3_W, fc3_b):
    Wi = h2.shape[1]
    Wo = W_rel.shape[1]
    NCLS = fc3_W.shape[1]

    def body(a_ref, h_ref, wr_ref, b_ref, wt_ref, bat_ref,
             w1_ref, b1_ref, w2_ref, b2_ref, w3_ref, b3_ref,
             o_ref, sum_ref, cnt_ref):
        i = pl.program_id(0)

        @pl.when(i == 0)
        def _():
            sum_ref[...] = jnp.zeros_like(sum_ref)
            cnt_ref[...] = jnp.zeros_like(cnt_ref)

        a = a_ref[0] + a_ref[1]
        h3 = _elu(jnp.dot(a, wr_ref[...], preferred_element_type=jnp.float32)
                  + jnp.dot(h_ref[...], wt_ref[...], preferred_element_type=jnp.float32)
                  + b_ref[...])
        bat = bat_ref[0, 0, :]
        onehot = (bat[:, None] == lax.broadcasted_iota(jnp.int32, (ROWS, G), 1)
                  ).astype(jnp.float32)
        sum_ref[...] += lax.dot_general(
            onehot, h3, (((0,), (0,)), ((), ())),
            preferred_element_type=jnp.float32)
        cnt_ref[...] += jnp.broadcast_to(jnp.sum(onehot, axis=0)[:, None], (G, Wo))

        @pl.when(i == GRID - 1)
        def _():
            pooled = sum_ref[...] / jnp.maximum(cnt_ref[...], 1.0)
            z = _elu(jnp.dot(pooled, w1_ref[...],
                             preferred_element_type=jnp.float32) + b1_ref[...])
            z = _elu(jnp.dot(z, w2_ref[...],
                             preferred_element_type=jnp.float32) + b2_ref[...])
            z = (jnp.dot(z, w3_ref[...], preferred_element_type=jnp.float32)
                 + b3_ref[...])
            m = jnp.max(z, axis=1, keepdims=True)
            zs = z - m
            lse = jnp.log(jnp.sum(jnp.exp(zs), axis=1, keepdims=True))
            o_ref[...] = zs - lse

    return pl.pallas_call(
        body,
        grid=(GRID,),
        in_specs=[
            pl.BlockSpec((2, ROWS, Wi), lambda i: (0, i, 0)),
            pl.BlockSpec((ROWS, Wi), lambda i: (i, 0)),
            pl.BlockSpec((Wi, Wo), lambda i: (0, 0)),
            pl.BlockSpec((1, Wo), lambda i: (0, 0)),
            pl.BlockSpec((Wi, Wo), lambda i: (0, 0)),
            pl.BlockSpec((1, 1, ROWS), lambda i: (i, 0, 0)),
            pl.BlockSpec((Wo, Wo), lambda i: (0, 0)),
            pl.BlockSpec((1, Wo), lambda i: (0, 0)),
            pl.BlockSpec((Wo, 32), lambda i: (0, 0)),
            pl.BlockSpec((1, 32), lambda i: (0, 0)),
            pl.BlockSpec((32, NCLS), lambda i: (0, 0)),
            pl.BlockSpec((1, NCLS), lambda i: (0, 0)),
        ],
        out_specs=pl.BlockSpec((G, NCLS), lambda i: (0, 0)),
        out_shape=jax.ShapeDtypeStruct((G, NCLS), jnp.float32),
        scratch_shapes=[
            pltpu.VMEM((G, Wo), jnp.float32),
            pltpu.VMEM((G, Wo), jnp.float32),
        ],
    )(agg, h2, W_rel, b_rel, W_root, batch3d,
      fc1_W, fc1_b, fc2_W, fc2_b, fc3_W, fc3_b)


def kernel(x, edge_index, batch, W1_rel, b1_rel, W1_root, W2_rel, b2_rel,
           W2_root, W3_rel, b3_rel, W3_root, fc1_W, fc1_b, fc2_W, fc2_b,
           fc3_W, fc3_b):
    zeros32 = jnp.zeros((N, 32), jnp.float32)
    zeros64 = jnp.zeros((N, 64), jnp.float32)
    batch3d = batch.reshape(GRID, 1, ROWS)
    src = edge_index[0].reshape(NW, NCHUNK, CHUNK)
    dst = edge_index[1].reshape(NW, NCHUNK, CHUNK)

    p1, r1 = _proj2(x, W1_rel, W1_root)
    agg1 = _edge_agg(p1, src, dst, zeros32, 32)
    h1 = _combine1(agg1, r1, b1_rel.reshape(1, -1))
    agg2 = _edge_agg(h1, src, dst, zeros32, 32)
    h2 = _combine_mm(agg2, h1, W2_rel, b2_rel.reshape(1, -1), W2_root)
    agg3 = _edge_agg(h2, src, dst, zeros64, 64)
    return _final_pool_head(agg3, h2, W3_rel, b3_rel.reshape(1, -1),
                            W3_root, batch3d,
                            fc1_W, fc1_b.reshape(1, -1),
                            fc2_W, fc2_b.reshape(1, -1),
                            fc3_W, fc3_b.reshape(1, -1))


# speedup vs baseline: 1.1731x; 1.1731x over previous
import functools

import jax
import jax.numpy as jnp
from jax import lax
from jax.experimental import pallas as pl
from jax.experimental.pallas import tpu as pltpu
from jax.experimental.pallas import tpu_sc as plsc

N = 10000
E = 320000
G = 64

NC = 2
NS = 16
NW = NC * NS
EPW = E // NW
CHUNK = 125
NCHUNK = EPW // CHUNK
NBUF = 8
LEAD = 4
NGRP = NCHUNK // NBUF
RZ = 624
RTAIL = N - NS * RZ

ROWS = 10000
GRID = N // ROWS


def _edge_agg(table, src, dst, zeros, W):
    mesh = plsc.VectorSubcoreMesh(core_axis_name="c", subcore_axis_name="s")

    @functools.partial(
        pl.kernel,
        out_type=jax.ShapeDtypeStruct((NC, N, W), jnp.float32),
        mesh=mesh,
        scratch_types=[
            pltpu.VMEM((NCHUNK, CHUNK), jnp.int32),
            pltpu.VMEM((NCHUNK, CHUNK), jnp.int32),
            [pltpu.VMEM((CHUNK, W), jnp.float32) for _ in range(NBUF)],
            pltpu.VMEM_SHARED((N, W), jnp.float32),
            [pltpu.SemaphoreType.DMA for _ in range(NBUF)],
            [pltpu.SemaphoreType.DMA for _ in range(NBUF)],
        ],
        compiler_params=pltpu.CompilerParams(use_tc_tiling_on_sc=False),
    )
    def k(tab_hbm, src_hbm, dst_hbm, zero_hbm, out_hbm, sidx, didx, rows,
          acc, gsem, ssem):
        c = lax.axis_index("c")
        s = lax.axis_index("s")
        wid = c * NS + s

        pltpu.sync_copy(src_hbm.at[wid], sidx)
        pltpu.sync_copy(dst_hbm.at[wid], didx)
        for b in range(LEAD):
            pltpu.async_copy(tab_hbm.at[sidx.at[b]], rows[b], gsem[b])

        pltpu.sync_copy(zero_hbm.at[pl.ds(s * RZ, RZ)], acc.at[pl.ds(s * RZ, RZ)])

        @pl.when(s == NS - 1)
        def _():
            pltpu.sync_copy(zero_hbm.at[pl.ds(NS * RZ, RTAIL)],
                            acc.at[pl.ds(NS * RZ, RTAIL)])

        plsc.subcore_barrier()

        def body(g, carry):
            for b in range(NBUF):
                kk = g * NBUF + b
                pltpu.make_async_copy(tab_hbm.at[sidx.at[kk]], rows[b],
                                      gsem[b]).wait()
                pltpu.async_copy(rows[b], acc.at[didx.at[kk]], ssem[b],
                                 add=True)
                bl = (b + LEAD) % NBUF

                @pl.when(kk + LEAD < NCHUNK)
                def _(kk=kk, bl=bl):
                    @pl.when(kk - (NBUF - LEAD) >= 0)
                    def _():
                        kd = kk - (NBUF - LEAD)
                        pltpu.make_async_copy(rows[bl], acc.at[didx.at[kd]],
                                              ssem[bl]).wait()
                    pltpu.async_copy(tab_hbm.at[sidx.at[kk + LEAD]], rows[bl],
                                     gsem[bl])

            return carry

        lax.fori_loop(0, NGRP, body, 0)
        for b in range(NBUF):
            kd = NCHUNK - NBUF + b
            pltpu.make_async_copy(rows[b], acc.at[didx.at[kd]],
                                  ssem[b]).wait()
        plsc.subcore_barrier()
        pltpu.sync_copy(acc.at[pl.ds(s * RZ, RZ)], out_hbm.at[c, pl.ds(s * RZ, RZ)])

        @pl.when(s == NS - 1)
        def _():
            pltpu.sync_copy(acc.at[pl.ds(NS * RZ, RTAIL)],
                            out_hbm.at[c, pl.ds(NS * RZ, RTAIL)])

    return k(table, src, dst, zeros)


def _proj2(x4, Wa_blk, Wb_blk):
    PB = x4.shape[0]
    D4 = x4.shape[1]

    def body(x_ref, wa_ref, wb_ref, p_ref, r_ref):
        xb = x_ref[...]
        p_ref[...] = jnp.dot(xb, wa_ref[...], preferred_element_type=jnp.float32)
        r_ref[...] = jnp.dot(xb, wb_ref[...], preferred_element_type=jnp.float32)

    return pl.pallas_call(
        body,
        out_shape=[
            jax.ShapeDtypeStruct((PB, 128), jnp.float32),
            jax.ShapeDtypeStruct((PB, 128), jnp.float32),
        ],
    )(x4, Wa_blk, Wb_blk)


def _elu(v):
    return jnp.where(v > 0, v, jnp.exp(jnp.minimum(v, 0.0)) - 1.0)


def _combine1(aggpk, r1pk, b1_tiled):
    PB = r1pk.shape[0]

    def body(a0_ref, a1_ref, r_ref, b_ref, o_ref):
        o_ref[...] = _elu(a0_ref[0] + a1_ref[0] + r_ref[...] + b_ref[...])

    return pl.pallas_call(
        body,
        grid=(1,),
        in_specs=[
            pl.BlockSpec((1, PB, 128), lambda i: (0, 0, 0)),
            pl.BlockSpec((1, PB, 128), lambda i: (1, 0, 0)),
            pl.BlockSpec((PB, 128), lambda i: (0, 0)),
            pl.BlockSpec((1, 128), lambda i: (0, 0)),
        ],
        out_specs=pl.BlockSpec((PB, 128), lambda i: (0, 0)),
        out_shape=jax.ShapeDtypeStruct(r1pk.shape, jnp.float32),
    )(aggpk, aggpk, r1pk, b1_tiled)


def _combine_mm(aggpk, hpk, Wr_blk, b_tiled, Wt_blk):
    PB = hpk.shape[0]
    WO = Wr_blk.shape[1]

    def body(a0_ref, a1_ref, h_ref, wr_ref, b_ref, wt_ref, o_ref):
        a = a0_ref[0] + a1_ref[0]
        v = (jnp.dot(a, wr_ref[...], preferred_element_type=jnp.float32)
             + jnp.dot(h_ref[...], wt_ref[...], preferred_element_type=jnp.float32)
             + b_ref[...])
        o_ref[...] = _elu(v)

    return pl.pallas_call(
        body,
        grid=(1,),
        in_specs=[
            pl.BlockSpec((1, PB, 128), lambda i: (0, 0, 0)),
            pl.BlockSpec((1, PB, 128), lambda i: (1, 0, 0)),
            pl.BlockSpec((PB, 128), lambda i: (0, 0)),
            pl.BlockSpec((128, WO), lambda i: (0, 0)),
            pl.BlockSpec((1, WO), lambda i: (0, 0)),
            pl.BlockSpec((128, WO), lambda i: (0, 0)),
        ],
        out_specs=pl.BlockSpec((PB, WO), lambda i: (0, 0)),
        out_shape=jax.ShapeDtypeStruct((PB, WO), jnp.float32),
    )(aggpk, aggpk, hpk, Wr_blk, b_tiled, Wt_blk)


def _final_pool_head(aggpk, hpk2, Wr_blk, b_tiled, Wt_blk, bat_ev, bat_od,
                     fc1_W, fc1_b, fc2_W, fc2_b, fc3_W, fc3_b):
    PB = hpk2.shape[0]
    NCLS = fc3_W.shape[1]

    def body(a0_ref, a1_ref, h_ref, wr_ref, b_ref, wt_ref, be_ref, bo_ref,
             w1_ref, b1_ref, w2_ref, b2_ref, w3_ref, b3_ref, o_ref):
        a = a0_ref[0] + a1_ref[0]
        h3 = _elu(jnp.dot(a, wr_ref[...], preferred_element_type=jnp.float32)
                  + jnp.dot(h_ref[...], wt_ref[...],
                            preferred_element_type=jnp.float32)
                  + b_ref[...])
        iota = lax.broadcasted_iota(jnp.int32, (PB, G), 1)
        oh_e = (be_ref[0, 0, :][:, None] == iota).astype(jnp.float32)
        oh_o = (bo_ref[0, 0, :][:, None] == iota).astype(jnp.float32)
        dn = (((0,), (0,)), ((), ()))
        sums = (lax.dot_general(oh_e, h3[:, :64], dn,
                                preferred_element_type=jnp.float32)
                + lax.dot_general(oh_o, h3[:, 64:], dn,
                                  preferred_element_type=jnp.float32))
        counts = (jnp.sum(oh_e, axis=0) + jnp.sum(oh_o, axis=0))[:, None]
        pooled = sums / jnp.maximum(counts, 1.0)
        z = _elu(jnp.dot(pooled, w1_ref[...],
                         preferred_element_type=jnp.float32) + b1_ref[...])
        z = _elu(jnp.dot(z, w2_ref[...],
                         preferred_element_type=jnp.float32) + b2_ref[...])
        z = (jnp.dot(z, w3_ref[...], preferred_element_type=jnp.float32)
             + b3_ref[...])
        m = jnp.max(z, axis=1, keepdims=True)
        zs = z - m
        lse = jnp.log(jnp.sum(jnp.exp(zs), axis=1, keepdims=True))
        o_ref[...] = zs - lse

    return pl.pallas_call(
        body,
        grid=(1,),
        in_specs=[
            pl.BlockSpec((1, PB, 128), lambda i: (0, 0, 0)),
            pl.BlockSpec((1, PB, 128), lambda i: (1, 0, 0)),
            pl.BlockSpec((PB, 128), lambda i: (0, 0)),
            pl.BlockSpec((128, 128), lambda i: (0, 0)),
            pl.BlockSpec((1, 128), lambda i: (0, 0)),
            pl.BlockSpec((128, 128), lambda i: (0, 0)),
            pl.BlockSpec((1, 1, PB), lambda i: (0, 0, 0)),
            pl.BlockSpec((1, 1, PB), lambda i: (0, 0, 0)),
            pl.BlockSpec((64, 64), lambda i: (0, 0)),
            pl.BlockSpec((1, 64), lambda i: (0, 0)),
            pl.BlockSpec((64, 32), lambda i: (0, 0)),
            pl.BlockSpec((1, 32), lambda i: (0, 0)),
            pl.BlockSpec((32, NCLS), lambda i: (0, 0)),
            pl.BlockSpec((1, NCLS), lambda i: (0, 0)),
        ],
        out_specs=pl.BlockSpec((G, NCLS), lambda i: (0, 0)),
        out_shape=jax.ShapeDtypeStruct((G, NCLS), jnp.float32),
    )(aggpk, aggpk, hpk2, Wr_blk, b_tiled, Wt_blk, bat_ev, bat_od,
      fc1_W, fc1_b, fc2_W, fc2_b, fc3_W, fc3_b)


def kernel(x, edge_index, batch, W1_rel, b1_rel, W1_root, W2_rel, b2_rel,
           W2_root, W3_rel, b3_rel, W3_root, fc1_W, fc1_b, fc2_W, fc2_b,
           fc3_W, fc3_b):
    zeros32 = jnp.zeros((N, 32), jnp.float32)
    zeros64 = jnp.zeros((N, 64), jnp.float32)
    src = edge_index[0].reshape(NW, NCHUNK, CHUNK)
    dst = edge_index[1].reshape(NW, NCHUNK, CHUNK)
    eye4 = jnp.eye(4, dtype=jnp.float32)
    eye2 = jnp.eye(2, dtype=jnp.float32)

    x4 = x.reshape(N // 4, 512)
    p1pk, r1pk = _proj2(x4, jnp.kron(eye4, W1_rel), jnp.kron(eye4, W1_root))
    agg1 = _edge_agg(p1pk.reshape(N, 32), src, dst, zeros32, 32)
    h1pk = _combine1(agg1.reshape(NC, N * 32 // 128, 128), r1pk,
                     jnp.tile(b1_rel, 4).reshape(1, 128))
    agg2 = _edge_agg(h1pk.reshape(N, 32), src, dst, zeros32, 32)
    h2pk = _combine_mm(agg2.reshape(NC, N * 32 // 128, 128), h1pk,
                       jnp.kron(eye4, W2_rel),
                       jnp.tile(b2_rel, 4).reshape(1, 256),
                       jnp.kron(eye4, W2_root))
    agg3 = _edge_agg(h2pk.reshape(N, 64), src, dst, zeros64, 64)
    return _final_pool_head(agg3.reshape(NC, N * 64 // 128, 128),
                            h2pk.reshape(N // 2, 128),
                            jnp.kron(eye2, W3_rel),
                            jnp.tile(b3_rel, 2).reshape(1, 128),
                            jnp.kron(eye2, W3_root),
                            batch[0::2].reshape(1, 1, N // 2),
                            batch[1::2].reshape(1, 1, N // 2),
                            fc1_W, fc1_b.reshape(1, -1),
                            fc2_W, fc2_b.reshape(1, -1),
                            fc3_W, fc3_b.reshape(1, -1))


# speedup vs baseline: 1.2495x; 1.0652x over previous
import functools

import jax
import jax.numpy as jnp
from jax import lax
from jax.experimental import pallas as pl
from jax.experimental.pallas import tpu as pltpu
from jax.experimental.pallas import tpu_sc as plsc

N = 10000
E = 320000
G = 64

NC = 2
NS = 16
NW = NC * NS
EPW = E // NW
CHUNK = 125
NCHUNK = EPW // CHUNK
NBUF = 8
LEAD = 6
NGRP = NCHUNK // NBUF
RZ = 624
RTAIL = N - NS * RZ

ROWS = 10000
GRID = N // ROWS


def _edge_agg(table, src, dst, zeros, W):
    mesh = plsc.VectorSubcoreMesh(core_axis_name="c", subcore_axis_name="s")

    @functools.partial(
        pl.kernel,
        out_type=jax.ShapeDtypeStruct((NC, N, W), jnp.float32),
        mesh=mesh,
        scratch_types=[
            pltpu.VMEM((NCHUNK, CHUNK), jnp.int32),
            pltpu.VMEM((NCHUNK, CHUNK), jnp.int32),
            [pltpu.VMEM((CHUNK, W), jnp.float32) for _ in range(NBUF)],
            pltpu.VMEM_SHARED((N, W), jnp.float32),
            [pltpu.SemaphoreType.DMA for _ in range(NBUF)],
            [pltpu.SemaphoreType.DMA for _ in range(NBUF)],
        ],
        compiler_params=pltpu.CompilerParams(use_tc_tiling_on_sc=False),
    )
    def k(tab_hbm, src_hbm, dst_hbm, zero_hbm, out_hbm, sidx, didx, rows,
          acc, gsem, ssem):
        c = lax.axis_index("c")
        s = lax.axis_index("s")
        wid = c * NS + s

        pltpu.sync_copy(src_hbm.at[wid], sidx)
        pltpu.sync_copy(dst_hbm.at[wid], didx)
        for b in range(LEAD):
            pltpu.async_copy(tab_hbm.at[sidx.at[b]], rows[b], gsem[b])

        pltpu.sync_copy(zero_hbm.at[pl.ds(s * RZ, RZ)], acc.at[pl.ds(s * RZ, RZ)])

        @pl.when(s == NS - 1)
        def _():
            pltpu.sync_copy(zero_hbm.at[pl.ds(NS * RZ, RTAIL)],
                            acc.at[pl.ds(NS * RZ, RTAIL)])

        plsc.subcore_barrier()

        def body(g, carry):
            for b in range(NBUF):
                kk = g * NBUF + b
                pltpu.make_async_copy(tab_hbm.at[sidx.at[kk]], rows[b],
                                      gsem[b]).wait()
                pltpu.async_copy(rows[b], acc.at[didx.at[kk]], ssem[b],
                                 add=True)
                bl = (b + LEAD) % NBUF

                @pl.when(kk + LEAD < NCHUNK)
                def _(kk=kk, bl=bl):
                    @pl.when(kk - (NBUF - LEAD) >= 0)
                    def _():
                        kd = kk - (NBUF - LEAD)
                        pltpu.make_async_copy(rows[bl], acc.at[didx.at[kd]],
                                              ssem[bl]).wait()
                    pltpu.async_copy(tab_hbm.at[sidx.at[kk + LEAD]], rows[bl],
                                     gsem[bl])

            return carry

        lax.fori_loop(0, NGRP, body, 0)
        for b in range(NBUF):
            kd = NCHUNK - NBUF + b
            pltpu.make_async_copy(rows[b], acc.at[didx.at[kd]],
                                  ssem[b]).wait()
        plsc.subcore_barrier()
        pltpu.sync_copy(acc.at[pl.ds(s * RZ, RZ)], out_hbm.at[c, pl.ds(s * RZ, RZ)])

        @pl.when(s == NS - 1)
        def _():
            pltpu.sync_copy(acc.at[pl.ds(NS * RZ, RTAIL)],
                            out_hbm.at[c, pl.ds(NS * RZ, RTAIL)])

    return k(table, src, dst, zeros)


def _proj2(x4, Wa_blk, Wb_blk):
    PB = x4.shape[0]
    D4 = x4.shape[1]

    def body(x_ref, wa_ref, wb_ref, p_ref, r_ref):
        xb = x_ref[...]
        p_ref[...] = jnp.dot(xb, wa_ref[...], preferred_element_type=jnp.float32)
        r_ref[...] = jnp.dot(xb, wb_ref[...], preferred_element_type=jnp.float32)

    return pl.pallas_call(
        body,
        out_shape=[
            jax.ShapeDtypeStruct((PB, 128), jnp.float32),
            jax.ShapeDtypeStruct((PB, 128), jnp.float32),
        ],
    )(x4, Wa_blk, Wb_blk)


def _elu(v):
    return jnp.where(v > 0, v, jnp.exp(jnp.minimum(v, 0.0)) - 1.0)


def _combine1(aggpk, r1pk, b1_tiled):
    PB = r1pk.shape[0]

    def body(a0_ref, a1_ref, r_ref, b_ref, o_ref):
        o_ref[...] = _elu(a0_ref[0] + a1_ref[0] + r_ref[...] + b_ref[...])

    return pl.pallas_call(
        body,
        grid=(1,),
        in_specs=[
            pl.BlockSpec((1, PB, 128), lambda i: (0, 0, 0)),
            pl.BlockSpec((1, PB, 128), lambda i: (1, 0, 0)),
            pl.BlockSpec((PB, 128), lambda i: (0, 0)),
            pl.BlockSpec((1, 128), lambda i: (0, 0)),
        ],
        out_specs=pl.BlockSpec((PB, 128), lambda i: (0, 0)),
        out_shape=jax.ShapeDtypeStruct(r1pk.shape, jnp.float32),
    )(aggpk, aggpk, r1pk, b1_tiled)


def _combine_mm(aggpk, hpk, Wr_blk, b_tiled, Wt_blk):
    PB = hpk.shape[0]
    WO = Wr_blk.shape[1]

    def body(a0_ref, a1_ref, h_ref, wr_ref, b_ref, wt_ref, o_ref):
        a = a0_ref[0] + a1_ref[0]
        v = (jnp.dot(a, wr_ref[...], preferred_element_type=jnp.float32)
             + jnp.dot(h_ref[...], wt_ref[...], preferred_element_type=jnp.float32)
             + b_ref[...])
        o_ref[...] = _elu(v)

    return pl.pallas_call(
        body,
        grid=(1,),
        in_specs=[
            pl.BlockSpec((1, PB, 128), lambda i: (0, 0, 0)),
            pl.BlockSpec((1, PB, 128), lambda i: (1, 0, 0)),
            pl.BlockSpec((PB, 128), lambda i: (0, 0)),
            pl.BlockSpec((128, WO), lambda i: (0, 0)),
            pl.BlockSpec((1, WO), lambda i: (0, 0)),
            pl.BlockSpec((128, WO), lambda i: (0, 0)),
        ],
        out_specs=pl.BlockSpec((PB, WO), lambda i: (0, 0)),
        out_shape=jax.ShapeDtypeStruct((PB, WO), jnp.float32),
    )(aggpk, aggpk, hpk, Wr_blk, b_tiled, Wt_blk)


def _final_pool_head(aggpk, hpk2, Wr_blk, b_tiled, Wt_blk, bat_ev, bat_od,
                     fc1_W, fc1_b, fc2_W, fc2_b, fc3_W, fc3_b):
    PB = hpk2.shape[0]
    NCLS = fc3_W.shape[1]

    def body(a0_ref, a1_ref, h_ref, wr_ref, b_ref, wt_ref, be_ref, bo_ref,
             w1_ref, b1_ref, w2_ref, b2_ref, w3_ref, b3_ref, o_ref):
        a = a0_ref[0] + a1_ref[0]
        h3 = _elu(jnp.dot(a, wr_ref[...], preferred_element_type=jnp.float32)
                  + jnp.dot(h_ref[...], wt_ref[...],
                            preferred_element_type=jnp.float32)
                  + b_ref[...])
        iota = lax.broadcasted_iota(jnp.int32, (PB, G), 1)
        oh_e = (be_ref[0, 0, :][:, None] == iota).astype(jnp.float32)
        oh_o = (bo_ref[0, 0, :][:, None] == iota).astype(jnp.float32)
        dn = (((0,), (0,)), ((), ()))
        sums = (lax.dot_general(oh_e, h3[:, :64], dn,
                                preferred_element_type=jnp.float32)
                + lax.dot_general(oh_o, h3[:, 64:], dn,
                                  preferred_element_type=jnp.float32))
        counts = (jnp.sum(oh_e, axis=0) + jnp.sum(oh_o, axis=0))[:, None]
        pooled = sums / jnp.maximum(counts, 1.0)
        z = _elu(jnp.dot(pooled, w1_ref[...],
                         preferred_element_type=jnp.float32) + b1_ref[...])
        z = _elu(jnp.dot(z, w2_ref[...],
                         preferred_element_type=jnp.float32) + b2_ref[...])
        z = (jnp.dot(z, w3_ref[...], preferred_element_type=jnp.float32)
             + b3_ref[...])
        m = jnp.max(z, axis=1, keepdims=True)
        zs = z - m
        lse = jnp.log(jnp.sum(jnp.exp(zs), axis=1, keepdims=True))
        o_ref[...] = zs - lse

    return pl.pallas_call(
        body,
        grid=(1,),
        in_specs=[
            pl.BlockSpec((1, PB, 128), lambda i: (0, 0, 0)),
            pl.BlockSpec((1, PB, 128), lambda i: (1, 0, 0)),
            pl.BlockSpec((PB, 128), lambda i: (0, 0)),
            pl.BlockSpec((128, 128), lambda i: (0, 0)),
            pl.BlockSpec((1, 128), lambda i: (0, 0)),
            pl.BlockSpec((128, 128), lambda i: (0, 0)),
            pl.BlockSpec((1, 1, PB), lambda i: (0, 0, 0)),
            pl.BlockSpec((1, 1, PB), lambda i: (0, 0, 0)),
            pl.BlockSpec((64, 64), lambda i: (0, 0)),
            pl.BlockSpec((1, 64), lambda i: (0, 0)),
            pl.BlockSpec((64, 32), lambda i: (0, 0)),
            pl.BlockSpec((1, 32), lambda i: (0, 0)),
            pl.BlockSpec((32, NCLS), lambda i: (0, 0)),
            pl.BlockSpec((1, NCLS), lambda i: (0, 0)),
        ],
        out_specs=pl.BlockSpec((G, NCLS), lambda i: (0, 0)),
        out_shape=jax.ShapeDtypeStruct((G, NCLS), jnp.float32),
    )(aggpk, aggpk, hpk2, Wr_blk, b_tiled, Wt_blk, bat_ev, bat_od,
      fc1_W, fc1_b, fc2_W, fc2_b, fc3_W, fc3_b)


def kernel(x, edge_index, batch, W1_rel, b1_rel, W1_root, W2_rel, b2_rel,
           W2_root, W3_rel, b3_rel, W3_root, fc1_W, fc1_b, fc2_W, fc2_b,
           fc3_W, fc3_b):
    zeros32 = jnp.zeros((N, 32), jnp.float32)
    zeros64 = jnp.zeros((N, 64), jnp.float32)
    src = edge_index[0].reshape(NW, NCHUNK, CHUNK)
    dst = edge_index[1].reshape(NW, NCHUNK, CHUNK)
    eye4 = jnp.eye(4, dtype=jnp.float32)
    eye2 = jnp.eye(2, dtype=jnp.float32)

    x4 = x.reshape(N // 4, 512)
    p1pk, r1pk = _proj2(x4, jnp.kron(eye4, W1_rel), jnp.kron(eye4, W1_root))
    agg1 = _edge_agg(p1pk.reshape(N, 32), src, dst, zeros32, 32)
    h1pk = _combine1(agg1.reshape(NC, N * 32 // 128, 128), r1pk,
                     jnp.tile(b1_rel, 4).reshape(1, 128))
    agg2 = _edge_agg(h1pk.reshape(N, 32), src, dst, zeros32, 32)
    h2pk = _combine_mm(agg2.reshape(NC, N * 32 // 128, 128), h1pk,
                       jnp.kron(eye4, W2_rel),
                       jnp.tile(b2_rel, 4).reshape(1, 256),
                       jnp.kron(eye4, W2_root))
    agg3 = _edge_agg(h2pk.reshape(N, 64), src, dst, zeros64, 64)
    return _final_pool_head(agg3.reshape(NC, N * 64 // 128, 128),
                            h2pk.reshape(N // 2, 128),
                            jnp.kron(eye2, W3_rel),
                            jnp.tile(b3_rel, 2).reshape(1, 128),
                            jnp.kron(eye2, W3_root),
                            batch[0::2].reshape(1, 1, N // 2),
                            batch[1::2].reshape(1, 1, N // 2),
                            fc1_W, fc1_b.reshape(1, -1),
                            fc2_W, fc2_b.reshape(1, -1),
                            fc3_W, fc3_b.reshape(1, -1))


# speedup vs baseline: 1.2528x; 1.0026x over previous
import functools

import jax
import jax.numpy as jnp
from jax import lax
from jax.experimental import pallas as pl
from jax.experimental.pallas import tpu as pltpu
from jax.experimental.pallas import tpu_sc as plsc

N = 10000
E = 320000
G = 64

NC = 2
NS = 16
NW = NC * NS
EPW = E // NW
CHUNK = 125
NCHUNK = EPW // CHUNK
NBUF = 8
LEAD = 7
NGRP = NCHUNK // NBUF
RZ = 624
RTAIL = N - NS * RZ

ROWS = 10000
GRID = N // ROWS


def _edge_agg(table, src, dst, zeros, W):
    mesh = plsc.VectorSubcoreMesh(core_axis_name="c", subcore_axis_name="s")

    @functools.partial(
        pl.kernel,
        out_type=jax.ShapeDtypeStruct((NC, N, W), jnp.float32),
        mesh=mesh,
        scratch_types=[
            pltpu.VMEM((NCHUNK, CHUNK), jnp.int32),
            pltpu.VMEM((NCHUNK, CHUNK), jnp.int32),
            [pltpu.VMEM((CHUNK, W), jnp.float32) for _ in range(NBUF)],
            pltpu.VMEM_SHARED((N, W), jnp.float32),
            [pltpu.SemaphoreType.DMA for _ in range(NBUF)],
            [pltpu.SemaphoreType.DMA for _ in range(NBUF)],
        ],
        compiler_params=pltpu.CompilerParams(use_tc_tiling_on_sc=False),
    )
    def k(tab_hbm, src_hbm, dst_hbm, zero_hbm, out_hbm, sidx, didx, rows,
          acc, gsem, ssem):
        c = lax.axis_index("c")
        s = lax.axis_index("s")
        wid = c * NS + s

        pltpu.sync_copy(src_hbm.at[wid], sidx)
        pltpu.sync_copy(dst_hbm.at[wid], didx)
        for b in range(LEAD):
            pltpu.async_copy(tab_hbm.at[sidx.at[b]], rows[b], gsem[b])

        pltpu.sync_copy(zero_hbm.at[pl.ds(s * RZ, RZ)], acc.at[pl.ds(s * RZ, RZ)])

        @pl.when(s == NS - 1)
        def _():
            pltpu.sync_copy(zero_hbm.at[pl.ds(NS * RZ, RTAIL)],
                            acc.at[pl.ds(NS * RZ, RTAIL)])

        plsc.subcore_barrier()

        def body(g, carry):
            for b in range(NBUF):
                kk = g * NBUF + b
                pltpu.make_async_copy(tab_hbm.at[sidx.at[kk]], rows[b],
                                      gsem[b]).wait()
                pltpu.async_copy(rows[b], acc.at[didx.at[kk]], ssem[b],
                                 add=True)
                bl = (b + LEAD) % NBUF

                @pl.when(kk + LEAD < NCHUNK)
                def _(kk=kk, bl=bl):
                    @pl.when(kk - (NBUF - LEAD) >= 0)
                    def _():
                        kd = kk - (NBUF - LEAD)
                        pltpu.make_async_copy(rows[bl], acc.at[didx.at[kd]],
                                              ssem[bl]).wait()
                    pltpu.async_copy(tab_hbm.at[sidx.at[kk + LEAD]], rows[bl],
                                     gsem[bl])

            return carry

        lax.fori_loop(0, NGRP, body, 0)
        for b in range(NBUF):
            kd = NCHUNK - NBUF + b
            pltpu.make_async_copy(rows[b], acc.at[didx.at[kd]],
                                  ssem[b]).wait()
        plsc.subcore_barrier()
        pltpu.sync_copy(acc.at[pl.ds(s * RZ, RZ)], out_hbm.at[c, pl.ds(s * RZ, RZ)])

        @pl.when(s == NS - 1)
        def _():
            pltpu.sync_copy(acc.at[pl.ds(NS * RZ, RTAIL)],
                            out_hbm.at[c, pl.ds(NS * RZ, RTAIL)])

    return k(table, src, dst, zeros)


def _proj2(x4, Wa_blk, Wb_blk):
    PB = x4.shape[0]
    D4 = x4.shape[1]

    def body(x_ref, wa_ref, wb_ref, p_ref, r_ref):
        xb = x_ref[...]
        p_ref[...] = jnp.dot(xb, wa_ref[...], preferred_element_type=jnp.float32)
        r_ref[...] = jnp.dot(xb, wb_ref[...], preferred_element_type=jnp.float32)

    return pl.pallas_call(
        body,
        out_shape=[
            jax.ShapeDtypeStruct((PB, 128), jnp.float32),
            jax.ShapeDtypeStruct((PB, 128), jnp.float32),
        ],
    )(x4, Wa_blk, Wb_blk)


def _elu(v):
    return jnp.where(v > 0, v, jnp.exp(jnp.minimum(v, 0.0)) - 1.0)


def _combine1(aggpk, r1pk, b1_tiled):
    PB = r1pk.shape[0]

    def body(a0_ref, a1_ref, r_ref, b_ref, o_ref):
        o_ref[...] = _elu(a0_ref[0] + a1_ref[0] + r_ref[...] + b_ref[...])

    return pl.pallas_call(
        body,
        grid=(1,),
        in_specs=[
            pl.BlockSpec((1, PB, 128), lambda i: (0, 0, 0)),
            pl.BlockSpec((1, PB, 128), lambda i: (1, 0, 0)),
            pl.BlockSpec((PB, 128), lambda i: (0, 0)),
            pl.BlockSpec((1, 128), lambda i: (0, 0)),
        ],
        out_specs=pl.BlockSpec((PB, 128), lambda i: (0, 0)),
        out_shape=jax.ShapeDtypeStruct(r1pk.shape, jnp.float32),
    )(aggpk, aggpk, r1pk, b1_tiled)


def _combine_mm(aggpk, hpk, Wr_blk, b_tiled, Wt_blk):
    PB = hpk.shape[0]
    WO = Wr_blk.shape[1]

    def body(a0_ref, a1_ref, h_ref, wr_ref, b_ref, wt_ref, o_ref):
        a = a0_ref[0] + a1_ref[0]
        v = (jnp.dot(a, wr_ref[...], preferred_element_type=jnp.float32)
             + jnp.dot(h_ref[...], wt_ref[...], preferred_element_type=jnp.float32)
             + b_ref[...])
        o_ref[...] = _elu(v)

    return pl.pallas_call(
        body,
        grid=(1,),
        in_specs=[
            pl.BlockSpec((1, PB, 128), lambda i: (0, 0, 0)),
            pl.BlockSpec((1, PB, 128), lambda i: (1, 0, 0)),
            pl.BlockSpec((PB, 128), lambda i: (0, 0)),
            pl.BlockSpec((128, WO), lambda i: (0, 0)),
            pl.BlockSpec((1, WO), lambda i: (0, 0)),
            pl.BlockSpec((128, WO), lambda i: (0, 0)),
        ],
        out_specs=pl.BlockSpec((PB, WO), lambda i: (0, 0)),
        out_shape=jax.ShapeDtypeStruct((PB, WO), jnp.float32),
    )(aggpk, aggpk, hpk, Wr_blk, b_tiled, Wt_blk)


def _final_pool_head(aggpk, hpk2, Wr_blk, b_tiled, Wt_blk, bat_ev, bat_od,
                     fc1_W, fc1_b, fc2_W, fc2_b, fc3_W, fc3_b):
    PB = hpk2.shape[0]
    NCLS = fc3_W.shape[1]

    def body(a0_ref, a1_ref, h_ref, wr_ref, b_ref, wt_ref, be_ref, bo_ref,
             w1_ref, b1_ref, w2_ref, b2_ref, w3_ref, b3_ref, o_ref):
        a = a0_ref[0] + a1_ref[0]
        h3 = _elu(jnp.dot(a, wr_ref[...], preferred_element_type=jnp.float32)
                  + jnp.dot(h_ref[...], wt_ref[...],
                            preferred_element_type=jnp.float32)
                  + b_ref[...])
        iota = lax.broadcasted_iota(jnp.int32, (PB, G), 1)
        oh_e = (be_ref[0, 0, :][:, None] == iota).astype(jnp.float32)
        oh_o = (bo_ref[0, 0, :][:, None] == iota).astype(jnp.float32)
        dn = (((0,), (0,)), ((), ()))
        sums = (lax.dot_general(oh_e, h3[:, :64], dn,
                                preferred_element_type=jnp.float32)
                + lax.dot_general(oh_o, h3[:, 64:], dn,
                                  preferred_element_type=jnp.float32))
        counts = (jnp.sum(oh_e, axis=0) + jnp.sum(oh_o, axis=0))[:, None]
        pooled = sums / jnp.maximum(counts, 1.0)
        z = _elu(jnp.dot(pooled, w1_ref[...],
                         preferred_element_type=jnp.float32) + b1_ref[...])
        z = _elu(jnp.dot(z, w2_ref[...],
                         preferred_element_type=jnp.float32) + b2_ref[...])
        z = (jnp.dot(z, w3_ref[...], preferred_element_type=jnp.float32)
             + b3_ref[...])
        m = jnp.max(z, axis=1, keepdims=True)
        zs = z - m
        lse = jnp.log(jnp.sum(jnp.exp(zs), axis=1, keepdims=True))
        o_ref[...] = zs - lse

    return pl.pallas_call(
        body,
        grid=(1,),
        in_specs=[
            pl.BlockSpec((1, PB, 128), lambda i: (0, 0, 0)),
            pl.BlockSpec((1, PB, 128), lambda i: (1, 0, 0)),
            pl.BlockSpec((PB, 128), lambda i: (0, 0)),
            pl.BlockSpec((128, 128), lambda i: (0, 0)),
            pl.BlockSpec((1, 128), lambda i: (0, 0)),
            pl.BlockSpec((128, 128), lambda i: (0, 0)),
            pl.BlockSpec((1, 1, PB), lambda i: (0, 0, 0)),
            pl.BlockSpec((1, 1, PB), lambda i: (0, 0, 0)),
            pl.BlockSpec((64, 64), lambda i: (0, 0)),
            pl.BlockSpec((1, 64), lambda i: (0, 0)),
            pl.BlockSpec((64, 32), lambda i: (0, 0)),
            pl.BlockSpec((1, 32), lambda i: (0, 0)),
            pl.BlockSpec((32, NCLS), lambda i: (0, 0)),
            pl.BlockSpec((1, NCLS), lambda i: (0, 0)),
        ],
        out_specs=pl.BlockSpec((G, NCLS), lambda i: (0, 0)),
        out_shape=jax.ShapeDtypeStruct((G, NCLS), jnp.float32),
    )(aggpk, aggpk, hpk2, Wr_blk, b_tiled, Wt_blk, bat_ev, bat_od,
      fc1_W, fc1_b, fc2_W, fc2_b, fc3_W, fc3_b)


def kernel(x, edge_index, batch, W1_rel, b1_rel, W1_root, W2_rel, b2_rel,
           W2_root, W3_rel, b3_rel, W3_root, fc1_W, fc1_b, fc2_W, fc2_b,
           fc3_W, fc3_b):
    zeros32 = jnp.zeros((N, 32), jnp.float32)
    zeros64 = jnp.zeros((N, 64), jnp.float32)
    src = edge_index[0].reshape(NW, NCHUNK, CHUNK)
    dst = edge_index[1].reshape(NW, NCHUNK, CHUNK)
    eye4 = jnp.eye(4, dtype=jnp.float32)
    eye2 = jnp.eye(2, dtype=jnp.float32)

    x4 = x.reshape(N // 4, 512)
    p1pk, r1pk = _proj2(x4, jnp.kron(eye4, W1_rel), jnp.kron(eye4, W1_root))
    agg1 = _edge_agg(p1pk.reshape(N, 32), src, dst, zeros32, 32)
    h1pk = _combine1(agg1.reshape(NC, N * 32 // 128, 128), r1pk,
                     jnp.tile(b1_rel, 4).reshape(1, 128))
    agg2 = _edge_agg(h1pk.reshape(N, 32), src, dst, zeros32, 32)
    h2pk = _combine_mm(agg2.reshape(NC, N * 32 // 128, 128), h1pk,
                       jnp.kron(eye4, W2_rel),
                       jnp.tile(b2_rel, 4).reshape(1, 256),
                       jnp.kron(eye4, W2_root))
    agg3 = _edge_agg(h2pk.reshape(N, 64), src, dst, zeros64, 64)
    return _final_pool_head(agg3.reshape(NC, N * 64 // 128, 128),
                            h2pk.reshape(N // 2, 128),
                            jnp.kron(eye2, W3_rel),
                            jnp.tile(b3_rel, 2).reshape(1, 128),
                            jnp.kron(eye2, W3_root),
                            batch[0::2].reshape(1, 1, N // 2),
                            batch[1::2].reshape(1, 1, N // 2),
                            fc1_W, fc1_b.reshape(1, -1),
                            fc2_W, fc2_b.reshape(1, -1),
                            fc3_W, fc3_b.reshape(1, -1))
